# baseline (speedup 1.0000x reference)
import functools

import jax
import jax.numpy as jnp
from jax import lax
from jax.experimental import pallas as pl
from jax.experimental.pallas import tpu as pltpu
from jax.experimental.pallas import tpu_sc as plsc

_EPS1 = 1e-09
_BN_EPS = 1e-05

_NC = 2
_NS = 16
_NW = _NC * _NS
_CH = 128
_LANES = 16


def _chunk_sizes(total, step):
    out = []
    done = 0
    while done < total:
        out.append(min(step, total - done))
        done += step
    return out


@functools.lru_cache(maxsize=None)
def _make_spmm(n, d, e_pad):
    k_chunks = e_pad // (_NW * _CH)
    per_sub = (((n + _NS - 1) // _NS) + 7) // 8 * 8
    last_sub = n - per_sub * (_NS - 1)
    assert last_sub > 0 and per_sub % 8 == 0 and last_sub % 8 == 0
    n_sub_idx = _CH // 128

    assert k_chunks % 2 == 0 and _CH == 128 and n_sub_idx == 1
    epw = k_chunks * _CH

    mesh = plsc.VectorSubcoreMesh(core_axis_name="c", subcore_axis_name="s")

    @functools.partial(
        pl.kernel,
        mesh=mesh,
        out_type=jax.ShapeDtypeStruct((_NC, n, d), jnp.float32),
        scratch_types=[
            pltpu.VMEM((epw,), jnp.int32),
            pltpu.VMEM((2, 128), jnp.int32),
            pltpu.VMEM((2, 128), jnp.float32),
            pltpu.VMEM((_CH, d), jnp.float32),
            pltpu.VMEM((_CH, d), jnp.float32),
            pltpu.VMEM_SHARED((n, d), jnp.float32),
            pltpu.SemaphoreType.DMA,
            pltpu.SemaphoreType.DMA,
            pltpu.SemaphoreType.DMA,
            pltpu.SemaphoreType.DMA,
            pltpu.SemaphoreType.DMA,
            pltpu.SemaphoreType.DMA,
            pltpu.SemaphoreType.DMA,
        ],
    )
    def spmm(x_hbm, src_hbm, dst_hbm, w_hbm, out_hbm,
             src_v, dst_v, w_v, rows0, rows1, agg_sh,
             sem_ld, sem_g0, sem_g1, sem_s0, sem_s1, sem_w0, sem_w1):
        c = lax.axis_index("c")
        s = lax.axis_index("s")
        wid = s * _NC + c
        rows = (rows0, rows1)
        sem_g = (sem_g0, sem_g1)
        sem_s = (sem_s0, sem_s1)
        sem_wd = (sem_w0, sem_w1)

        ebase = wid * epw
        pltpu.async_copy(src_hbm.at[pl.ds(ebase, epw)], src_v, sem_ld)

        scope_zero = jax.named_scope("agg_zero")
        scope_zero.__enter__()
        zero = jnp.zeros((_LANES,), jnp.float32)

        def zbody(t, _):
            i = t // (d // _LANES)
            j = t % (d // _LANES)
            rows0[i, pl.ds(j * _LANES, _LANES)] = zero
            return _

        lax.fori_loop(0, _CH * (d // _LANES), zbody, None)

        zbase = s * per_sub

        def _blit_zeros(nrows):
            done = 0
            for step in _chunk_sizes(nrows, _CH):
                pltpu.sync_copy(rows0.at[pl.ds(0, step)],
                                agg_sh.at[pl.ds(zbase + done, step)])
                done += step

        @pl.when(s == _NS - 1)
        def _():
            _blit_zeros(last_sub)

        @pl.when(s != _NS - 1)
        def _():
            _blit_zeros(per_sub)

        plsc.subcore_barrier()
        scope_zero.__exit__(None, None, None)

        scope_edges = jax.named_scope("edge_loop")
        scope_edges.__enter__()
        pltpu.make_async_copy(src_hbm.at[pl.ds(ebase, epw)], src_v,
                              sem_ld).wait()

        def _gather(k, b):
            return pltpu.make_async_copy(
                x_hbm.at[c].at[src_v.at[pl.ds(k * _CH, _CH)]],
                rows[b], sem_g[b])

        def _scatter(k, b):
            return pltpu.make_async_copy(
                rows[b], agg_sh.at[dst_v.at[b]], sem_s[b])

        def _wd_loads(k, b):
            cid = wid * k_chunks + k
            return [
                pltpu.make_async_copy(
                    w_hbm.at[pl.ds(cid * _CH, _CH)], w_v.at[b], sem_wd[b]),
                pltpu.make_async_copy(dst_hbm.at[cid], dst_v.at[b],
                                      sem_wd[b]),
            ]

        def _scale(k, b):
            def srow(g, _):
                wv = w_v[b, pl.ds(g * _LANES, _LANES)]
                for e in range(_LANES):
                    i = g * _LANES + e
                    wgt = wv[e]
                    for j in range(d // _LANES):
                        sl = pl.ds(j * _LANES, _LANES)
                        rows[b][i, sl] = rows[b][i, sl] * wgt
                return _

            lax.fori_loop(0, _CH // _LANES, srow, None)

        _gather(0, 0).start()
        for cp in _wd_loads(0, 0):
            cp.start()

        def outer(t, _):
            for b in range(2):
                k = 2 * t + b

                @pl.when(k >= 1)
                def _():
                    _scatter(k - 1, 1 - b).wait()

                @pl.when(k < k_chunks - 1)
                def _():
                    _gather(k + 1, 1 - b).start()
                    for cp in _wd_loads(k + 1, 1 - b):
                        cp.start()

                _gather(k, b).wait()
                for cp in _wd_loads(k, b):
                    cp.wait()
                _scale(k, b)
                _scatter(k, b).start(add=True)
            return _

        lax.fori_loop(0, k_chunks // 2, outer, None)
        _scatter(k_chunks - 1, 1).wait()
        plsc.subcore_barrier()
        scope_edges.__exit__(None, None, None)

        scope_out = jax.named_scope("agg_writeout")
        scope_out.__enter__()
        obase = s * per_sub

        def _writeout(nrows):
            done = 0
            for step in _chunk_sizes(nrows, _CH):
                pltpu.sync_copy(agg_sh.at[pl.ds(obase + done, step)],
                                out_hbm.at[c, pl.ds(obase + done, step)])
                done += step

        @pl.when(s == _NS - 1)
        def _():
            _writeout(last_sub)

        @pl.when(s != _NS - 1)
        def _():
            _writeout(per_sub)

        scope_out.__exit__(None, None, None)

    return spmm


def _mlp_body(x_ref, agg_ref, w1_ref, b1_ref, g_ref, be_ref, w2_ref, b2_ref,
              o_ref):
    x = x_ref[...]
    h = (1.0 + _EPS1) * x + agg_ref[0] + agg_ref[1]
    z = lax.dot_general(h, w1_ref[...], (((1,), (1,)), ((), ())),
                        preferred_element_type=jnp.float32) + b1_ref[...]
    mean = jnp.mean(z, axis=0, keepdims=True)
    var = jnp.mean((z - mean) ** 2, axis=0, keepdims=True)
    zn = (z - mean) * lax.rsqrt(var + _BN_EPS) * g_ref[...] + be_ref[...]
    r = jnp.maximum(zn, 0.0)
    o_ref[...] = lax.dot_general(r, w2_ref[...], (((1,), (1,)), ((), ())),
                                 preferred_element_type=jnp.float32) + b2_ref[...]


@functools.lru_cache(maxsize=None)
def _make_mlp(n, d, h_dim, o_dim):
    return pl.pallas_call(
        _mlp_body,
        out_shape=jax.ShapeDtypeStruct((n, o_dim), jnp.float32),
    )


def kernel(x, edge_index, edge_weight, W1, b1, gamma, beta, W2, b2):
    n, d = x.shape
    e = edge_weight.shape[0]
    per_round = _NW * _CH * 2
    e_pad = ((e + per_round - 1) // per_round) * per_round
    pad = e_pad - e

    dst = edge_index[0]
    src = edge_index[1]
    if pad:
        src = jnp.concatenate([src, jnp.zeros((pad,), jnp.int32)])
        dst = jnp.concatenate([dst, jnp.zeros((pad,), jnp.int32)])
        w = jnp.concatenate([edge_weight, jnp.zeros((pad,), jnp.float32)])
    else:
        w = edge_weight
    dst2d = dst.reshape(e_pad // 128, 128)

    x2 = jnp.concatenate([x[None], x[None]], axis=0)
    agg = _make_spmm(n, d, e_pad)(x2, src, dst2d, w)

    mlp = _make_mlp(n, d, W1.shape[0], W2.shape[0])
    return mlp(x, agg, W1, b1.reshape(1, -1), gamma.reshape(1, -1),
               beta.reshape(1, -1), W2, b2.reshape(1, -1))


# speedup vs baseline: 2.9799x; 2.9799x over previous
import functools

import jax
import jax.numpy as jnp
from jax import lax
from jax.experimental import pallas as pl
from jax.experimental.pallas import tpu as pltpu
from jax.experimental.pallas import tpu_sc as plsc

_EPS1 = 1e-09
_BN_EPS = 1e-05

_NC = 2
_NS = 16
_NW = _NC * _NS
_CH = 128
_LANES = 16


def _chunk_sizes(total, step):
    out = []
    done = 0
    while done < total:
        out.append(min(step, total - done))
        done += step
    return out


@functools.lru_cache(maxsize=None)
def _make_spmm(n, d, e_pad):
    k_chunks = e_pad // (_NW * _CH)
    per_sub = (((n + _NS - 1) // _NS) + 7) // 8 * 8
    last_sub = n - per_sub * (_NS - 1)
    assert last_sub > 0 and per_sub % 8 == 0 and last_sub % 8 == 0
    n_sub_idx = _CH // 128

    assert k_chunks % 2 == 0 and _CH == 128 and n_sub_idx == 1
    epw = k_chunks * _CH

    mesh = plsc.VectorSubcoreMesh(core_axis_name="c", subcore_axis_name="s")

    @functools.partial(
        pl.kernel,
        mesh=mesh,
        out_type=jax.ShapeDtypeStruct((_NC, n, d), jnp.float32),
        scratch_types=[
            pltpu.VMEM((epw,), jnp.int32),
            pltpu.VMEM((2, 128), jnp.int32),
            pltpu.VMEM((2, 128), jnp.float32),
            pltpu.VMEM((_CH, d), jnp.float32),
            pltpu.VMEM((_CH, d), jnp.float32),
            pltpu.VMEM_SHARED((n, d), jnp.float32),
            pltpu.SemaphoreType.DMA,
            pltpu.SemaphoreType.DMA,
            pltpu.SemaphoreType.DMA,
            pltpu.SemaphoreType.DMA,
            pltpu.SemaphoreType.DMA,
            pltpu.SemaphoreType.DMA,
            pltpu.SemaphoreType.DMA,
        ],
    )
    def spmm(x_hbm, src_hbm, dst_hbm, w_hbm, out_hbm,
             src_v, dst_v, w_v, rows0, rows1, agg_sh,
             sem_ld, sem_g0, sem_g1, sem_s0, sem_s1, sem_w0, sem_w1):
        c = lax.axis_index("c")
        s = lax.axis_index("s")
        wid = s * _NC + c
        rows = (rows0, rows1)
        sem_g = (sem_g0, sem_g1)
        sem_s = (sem_s0, sem_s1)
        sem_wd = (sem_w0, sem_w1)

        ebase = wid * epw
        pltpu.async_copy(src_hbm.at[pl.ds(ebase, epw)], src_v, sem_ld)

        scope_zero = jax.named_scope("agg_zero")
        scope_zero.__enter__()
        zero = jnp.zeros((_LANES,), jnp.float32)

        def zbody(t, _):
            i = t // (d // _LANES)
            j = t % (d // _LANES)
            rows0[i, pl.ds(j * _LANES, _LANES)] = zero
            return _

        lax.fori_loop(0, _CH * (d // _LANES), zbody, None)

        zbase = s * per_sub

        def _blit_zeros(nrows):
            done = 0
            for step in _chunk_sizes(nrows, _CH):
                pltpu.sync_copy(rows0.at[pl.ds(0, step)],
                                agg_sh.at[pl.ds(zbase + done, step)])
                done += step

        @pl.when(s == _NS - 1)
        def _():
            _blit_zeros(last_sub)

        @pl.when(s != _NS - 1)
        def _():
            _blit_zeros(per_sub)

        plsc.subcore_barrier()
        scope_zero.__exit__(None, None, None)

        scope_edges = jax.named_scope("edge_loop")
        scope_edges.__enter__()
        pltpu.make_async_copy(src_hbm.at[pl.ds(ebase, epw)], src_v,
                              sem_ld).wait()

        def _gather(k, b):
            return pltpu.make_async_copy(
                x_hbm.at[c].at[src_v.at[pl.ds(k * _CH, _CH)]],
                rows[b], sem_g[b])

        def _scatter(k, b):
            return pltpu.make_async_copy(
                rows[b], agg_sh.at[dst_v.at[b]], sem_s[b])

        def _wd_loads(k, b):
            cid = wid * k_chunks + k
            return [
                pltpu.make_async_copy(
                    w_hbm.at[pl.ds(cid * _CH, _CH)], w_v.at[b], sem_wd[b]),
                pltpu.make_async_copy(dst_hbm.at[cid], dst_v.at[b],
                                      sem_wd[b]),
            ]

        def _scale(k, b):
            def srow(g, _):
                wv = w_v[b, pl.ds(g * _LANES, _LANES)]
                for e in range(_LANES):
                    i = g * _LANES + e
                    wgt = wv[e]
                    for j in range(d // _LANES):
                        sl = pl.ds(j * _LANES, _LANES)
                        rows[b][i, sl] = rows[b][i, sl] * wgt
                return _

            lax.fori_loop(0, _CH // _LANES, srow, None)

        def outer(t, _):
            for b in range(2):
                k = 2 * t + b

                @pl.when(k >= 1)
                def _():
                    _scatter(k - 1, 1 - b).wait()

                @pl.when(k < k_chunks - 1)
                def _():
                    _gather(k + 1, 1 - b).start()
                    for cp in _wd_loads(k + 1, 1 - b):
                        cp.start()

                _gather(k, b).wait()
                for cp in _wd_loads(k, b):
                    cp.wait()
                _scale(k, b)
                _scatter(k, b).start(add=True)
            return _

        _gather(0, 0).start()
        for cp in _wd_loads(0, 0):
            cp.start()
        lax.fori_loop(0, k_chunks // 2, outer, None)
        _scatter(k_chunks - 1, 1).wait()
        plsc.subcore_barrier()
        scope_edges.__exit__(None, None, None)

        scope_out = jax.named_scope("agg_writeout")
        scope_out.__enter__()
        obase = s * per_sub

        def _writeout(nrows):
            done = 0
            for step in _chunk_sizes(nrows, _CH):
                pltpu.sync_copy(agg_sh.at[pl.ds(obase + done, step)],
                                out_hbm.at[c, pl.ds(obase + done, step)])
                done += step

        @pl.when(s == _NS - 1)
        def _():
            _writeout(last_sub)

        @pl.when(s != _NS - 1)
        def _():
            _writeout(per_sub)

        scope_out.__exit__(None, None, None)

    return spmm


def _mlp_body(x_ref, agg_ref, w1_ref, b1_ref, g_ref, be_ref, w2_ref, b2_ref,
              o_ref):
    x = x_ref[...]
    h = (1.0 + _EPS1) * x + agg_ref[0] + agg_ref[1]
    z = lax.dot_general(h, w1_ref[...], (((1,), (1,)), ((), ())),
                        preferred_element_type=jnp.float32) + b1_ref[...]
    mean = jnp.mean(z, axis=0, keepdims=True)
    var = jnp.mean((z - mean) ** 2, axis=0, keepdims=True)
    zn = (z - mean) * lax.rsqrt(var + _BN_EPS) * g_ref[...] + be_ref[...]
    r = jnp.maximum(zn, 0.0)
    o_ref[...] = lax.dot_general(r, w2_ref[...], (((1,), (1,)), ((), ())),
                                 preferred_element_type=jnp.float32) + b2_ref[...]


@functools.lru_cache(maxsize=None)
def _make_mlp(n, d, h_dim, o_dim):
    return pl.pallas_call(
        _mlp_body,
        out_shape=jax.ShapeDtypeStruct((n, o_dim), jnp.float32),
    )


def kernel(x, edge_index, edge_weight, W1, b1, gamma, beta, W2, b2):
    n, d = x.shape
    e = edge_weight.shape[0]
    per_round = _NW * _CH * 2
    e_pad = ((e + per_round - 1) // per_round) * per_round
    pad = e_pad - e

    dst = edge_index[0]
    src = edge_index[1]
    if pad:
        fill = jnp.arange(pad, dtype=jnp.int32) % n
        src = jnp.concatenate([src, fill])
        dst = jnp.concatenate([dst, fill])
        w = jnp.concatenate([edge_weight, jnp.zeros((pad,), jnp.float32)])
    else:
        w = edge_weight
    dst2d = dst.reshape(e_pad // 128, 128)

    x2 = jnp.concatenate([x[None], x[None]], axis=0)
    agg = _make_spmm(n, d, e_pad)(x2, src, dst2d, w)

    mlp = _make_mlp(n, d, W1.shape[0], W2.shape[0])
    return mlp(x, agg, W1, b1.reshape(1, -1), gamma.reshape(1, -1),
               beta.reshape(1, -1), W2, b2.reshape(1, -1))


# speedup vs baseline: 3.0538x; 1.0248x over previous
import functools

import jax
import jax.numpy as jnp
from jax import lax
from jax.experimental import pallas as pl
from jax.experimental.pallas import tpu as pltpu
from jax.experimental.pallas import tpu_sc as plsc

_EPS1 = 1e-09
_BN_EPS = 1e-05

_NC = 2
_NS = 16
_NW = _NC * _NS
_CH = 128
_LANES = 16


def _chunk_sizes(total, step):
    out = []
    done = 0
    while done < total:
        out.append(min(step, total - done))
        done += step
    return out


@functools.lru_cache(maxsize=None)
def _make_spmm(n, d, e_pad):
    k_chunks = e_pad // (_NW * _CH)
    per_sub = (((n + _NS - 1) // _NS) + 7) // 8 * 8
    last_sub = n - per_sub * (_NS - 1)
    assert last_sub > 0 and per_sub % 8 == 0 and last_sub % 8 == 0
    n_sub_idx = _CH // 128

    assert k_chunks % 2 == 0 and _CH == 128 and n_sub_idx == 1
    epw = k_chunks * _CH

    mesh = plsc.VectorSubcoreMesh(core_axis_name="c", subcore_axis_name="s")

    @functools.partial(
        pl.kernel,
        mesh=mesh,
        out_type=jax.ShapeDtypeStruct((_NC, n, d), jnp.float32),
        scratch_types=[
            pltpu.VMEM((epw,), jnp.int32),
            pltpu.VMEM((2, 128), jnp.int32),
            pltpu.VMEM((2, 128), jnp.float32),
            pltpu.VMEM((_CH, d), jnp.float32),
            pltpu.VMEM((_CH, d), jnp.float32),
            pltpu.VMEM_SHARED((n, d), jnp.float32),
            pltpu.SemaphoreType.DMA,
            pltpu.SemaphoreType.DMA,
            pltpu.SemaphoreType.DMA,
            pltpu.SemaphoreType.DMA,
            pltpu.SemaphoreType.DMA,
            pltpu.SemaphoreType.DMA,
            pltpu.SemaphoreType.DMA,
        ],
    )
    def spmm(x_hbm, src_hbm, dst_hbm, w_hbm, out_hbm,
             src_v, dst_v, w_v, rows0, rows1, agg_sh,
             sem_ld, sem_g0, sem_g1, sem_s0, sem_s1, sem_w0, sem_w1):
        c = lax.axis_index("c")
        s = lax.axis_index("s")
        wid = s * _NC + c
        rows = (rows0, rows1)
        sem_g = (sem_g0, sem_g1)
        sem_s = (sem_s0, sem_s1)
        sem_wd = (sem_w0, sem_w1)

        ebase = wid * epw
        pltpu.async_copy(src_hbm.at[pl.ds(ebase, epw)], src_v, sem_ld)

        scope_zero = jax.named_scope("agg_zero")
        scope_zero.__enter__()
        zero = jnp.zeros((_LANES,), jnp.float32)

        def zbody(t, _):
            i = t // (d // _LANES)
            j = t % (d // _LANES)
            rows0[i, pl.ds(j * _LANES, _LANES)] = zero
            return _

        lax.fori_loop(0, _CH * (d // _LANES), zbody, None)

        zbase = s * per_sub

        def _blit_zeros(nrows):
            done = 0
            for step in _chunk_sizes(nrows, _CH):
                pltpu.sync_copy(rows0.at[pl.ds(0, step)],
                                agg_sh.at[pl.ds(zbase + done, step)])
                done += step

        @pl.when(s == _NS - 1)
        def _():
            _blit_zeros(last_sub)

        @pl.when(s != _NS - 1)
        def _():
            _blit_zeros(per_sub)

        plsc.subcore_barrier()
        scope_zero.__exit__(None, None, None)

        scope_edges = jax.named_scope("edge_loop")
        scope_edges.__enter__()
        pltpu.make_async_copy(src_hbm.at[pl.ds(ebase, epw)], src_v,
                              sem_ld).wait()

        def _gather(k, b):
            return pltpu.make_async_copy(
                x_hbm.at[src_v.at[pl.ds(k * _CH, _CH)]], rows[b], sem_g[b])

        def _scatter(k, b):
            return pltpu.make_async_copy(
                rows[b], agg_sh.at[dst_v.at[b]], sem_s[b])

        def _wd_loads(k, b):
            cid = wid * k_chunks + k
            return [
                pltpu.make_async_copy(
                    w_hbm.at[pl.ds(cid * _CH, _CH)], w_v.at[b], sem_wd[b]),
                pltpu.make_async_copy(dst_hbm.at[cid], dst_v.at[b],
                                      sem_wd[b]),
            ]

        def _scale(k, b):
            def srow(g, _):
                wv = w_v[b, pl.ds(g * _LANES, _LANES)]
                for e in range(_LANES):
                    i = g * _LANES + e
                    wgt = wv[e]
                    for j in range(d // _LANES):
                        sl = pl.ds(j * _LANES, _LANES)
                        rows[b][i, sl] = rows[b][i, sl] * wgt
                return _

            lax.fori_loop(0, _CH // _LANES, srow, None)

        def outer(t, _):
            for b in range(2):
                k = 2 * t + b

                @pl.when(k >= 1)
                def _():
                    _scatter(k - 1, 1 - b).wait()

                @pl.when(k < k_chunks - 1)
                def _():
                    _gather(k + 1, 1 - b).start()
                    for cp in _wd_loads(k + 1, 1 - b):
                        cp.start()

                _gather(k, b).wait()
                for cp in _wd_loads(k, b):
                    cp.wait()
                _scale(k, b)
                _scatter(k, b).start(add=True)
            return _

        _gather(0, 0).start()
        for cp in _wd_loads(0, 0):
            cp.start()
        lax.fori_loop(0, k_chunks // 2, outer, None)
        _scatter(k_chunks - 1, 1).wait()
        plsc.subcore_barrier()
        scope_edges.__exit__(None, None, None)

        scope_out = jax.named_scope("agg_writeout")
        scope_out.__enter__()
        obase = s * per_sub

        def _writeout(nrows):
            done = 0
            for step in _chunk_sizes(nrows, _CH):
                pltpu.sync_copy(agg_sh.at[pl.ds(obase + done, step)],
                                out_hbm.at[c, pl.ds(obase + done, step)])
                done += step

        @pl.when(s == _NS - 1)
        def _():
            _writeout(last_sub)

        @pl.when(s != _NS - 1)
        def _():
            _writeout(per_sub)

        scope_out.__exit__(None, None, None)

    return spmm


def _mlp_body(x_ref, agg_ref, w1_ref, b1_ref, g_ref, be_ref, w2_ref, b2_ref,
              o_ref):
    x = x_ref[...]
    h = (1.0 + _EPS1) * x + agg_ref[0] + agg_ref[1]
    z = lax.dot_general(h, w1_ref[...], (((1,), (1,)), ((), ())),
                        preferred_element_type=jnp.float32) + b1_ref[...]
    mean = jnp.mean(z, axis=0, keepdims=True)
    var = jnp.mean((z - mean) ** 2, axis=0, keepdims=True)
    zn = (z - mean) * lax.rsqrt(var + _BN_EPS) * g_ref[...] + be_ref[...]
    r = jnp.maximum(zn, 0.0)
    o_ref[...] = lax.dot_general(r, w2_ref[...], (((1,), (1,)), ((), ())),
                                 preferred_element_type=jnp.float32) + b2_ref[...]


@functools.lru_cache(maxsize=None)
def _make_mlp(n, d, h_dim, o_dim):
    return pl.pallas_call(
        _mlp_body,
        out_shape=jax.ShapeDtypeStruct((n, o_dim), jnp.float32),
    )


def kernel(x, edge_index, edge_weight, W1, b1, gamma, beta, W2, b2):
    n, d = x.shape
    e = edge_weight.shape[0]
    per_round = _NW * _CH * 2
    e_pad = ((e + per_round - 1) // per_round) * per_round
    pad = e_pad - e

    dst = edge_index[0]
    src = edge_index[1]
    if pad:
        fill = jnp.arange(pad, dtype=jnp.int32) % n
        src = jnp.concatenate([src, fill])
        dst = jnp.concatenate([dst, fill])
        w = jnp.concatenate([edge_weight, jnp.zeros((pad,), jnp.float32)])
    else:
        w = edge_weight
    dst2d = dst.reshape(e_pad // 128, 128)

    agg = _make_spmm(n, d, e_pad)(x, src, dst2d, w)

    mlp = _make_mlp(n, d, W1.shape[0], W2.shape[0])
    return mlp(x, agg, W1, b1.reshape(1, -1), gamma.reshape(1, -1),
               beta.reshape(1, -1), W2, b2.reshape(1, -1))


# speedup vs baseline: 3.0750x; 1.0070x over previous
import functools

import jax
import jax.numpy as jnp
from jax import lax
from jax.experimental import pallas as pl
from jax.experimental.pallas import tpu as pltpu
from jax.experimental.pallas import tpu_sc as plsc

_EPS1 = 1e-09
_BN_EPS = 1e-05

_NC = 2
_NS = 16
_NW = _NC * _NS
_CH = 128
_LANES = 16


def _chunk_sizes(total, step):
    out = []
    done = 0
    while done < total:
        out.append(min(step, total - done))
        done += step
    return out


@functools.lru_cache(maxsize=None)
def _make_spmm(n, d, e):
    per_sub = (((n + _NS - 1) // _NS) + 7) // 8 * 8
    last_sub = n - per_sub * (_NS - 1)
    assert last_sub > 0 and per_sub % 8 == 0 and last_sub % 8 == 0
    assert _CH == 128 and e % _CH == 0

    total_chunks = e // _CH
    kc_full = ((total_chunks + _NW - 1) // _NW + 1) // 2 * 2
    kcs = [max(0, min(kc_full, total_chunks - w * kc_full)) for w in range(_NW)]
    assert all(k >= 2 and k % 2 == 0 for k in kcs)
    part_w = min((w for w in range(_NW) if kcs[w] < kc_full), default=None)
    part_kc = kcs[part_w] if part_w is not None else kc_full
    k_chunks = kc_full
    epw = kc_full * _CH

    mesh = plsc.VectorSubcoreMesh(core_axis_name="c", subcore_axis_name="s")

    @functools.partial(
        pl.kernel,
        mesh=mesh,
        out_type=jax.ShapeDtypeStruct((_NC, n, d), jnp.float32),
        scratch_types=[
            pltpu.VMEM((epw,), jnp.int32),
            pltpu.VMEM((2, 128), jnp.int32),
            pltpu.VMEM((2, 128), jnp.float32),
            pltpu.VMEM((_CH, d), jnp.float32),
            pltpu.VMEM((_CH, d), jnp.float32),
            pltpu.VMEM_SHARED((n, d), jnp.float32),
            pltpu.SemaphoreType.DMA,
            pltpu.SemaphoreType.DMA,
            pltpu.SemaphoreType.DMA,
            pltpu.SemaphoreType.DMA,
            pltpu.SemaphoreType.DMA,
            pltpu.SemaphoreType.DMA,
            pltpu.SemaphoreType.DMA,
        ],
    )
    def spmm(x_hbm, src_hbm, dst_hbm, w_hbm, out_hbm,
             src_v, dst_v, w_v, rows0, rows1, agg_sh,
             sem_ld, sem_g0, sem_g1, sem_s0, sem_s1, sem_w0, sem_w1):
        c = lax.axis_index("c")
        s = lax.axis_index("s")
        wid = s * _NC + c
        rows = (rows0, rows1)
        sem_g = (sem_g0, sem_g1)
        sem_s = (sem_s0, sem_s1)
        sem_wd = (sem_w0, sem_w1)

        kc_w = jnp.clip(total_chunks - wid * kc_full, 0, kc_full)
        ebase = wid * epw
        if part_w is None:
            pltpu.async_copy(src_hbm.at[pl.ds(ebase, epw)], src_v, sem_ld)
        else:
            @pl.when(wid != part_w)
            def _():
                pltpu.async_copy(src_hbm.at[pl.ds(ebase, epw)], src_v,
                                 sem_ld)

            @pl.when(wid == part_w)
            def _():
                pltpu.async_copy(
                    src_hbm.at[pl.ds(ebase, part_kc * _CH)],
                    src_v.at[pl.ds(0, part_kc * _CH)], sem_ld)

        scope_zero = jax.named_scope("agg_zero")
        scope_zero.__enter__()
        zero = jnp.zeros((_LANES,), jnp.float32)

        def zbody(t, _):
            i = t // (d // _LANES)
            j = t % (d // _LANES)
            rows0[i, pl.ds(j * _LANES, _LANES)] = zero
            return _

        lax.fori_loop(0, _CH * (d // _LANES), zbody, None)

        zbase = s * per_sub

        def _blit_zeros(nrows):
            done = 0
            for step in _chunk_sizes(nrows, _CH):
                pltpu.sync_copy(rows0.at[pl.ds(0, step)],
                                agg_sh.at[pl.ds(zbase + done, step)])
                done += step

        @pl.when(s == _NS - 1)
        def _():
            _blit_zeros(last_sub)

        @pl.when(s != _NS - 1)
        def _():
            _blit_zeros(per_sub)

        plsc.subcore_barrier()
        scope_zero.__exit__(None, None, None)

        scope_edges = jax.named_scope("edge_loop")
        scope_edges.__enter__()
        if part_w is None:
            pltpu.make_async_copy(src_hbm.at[pl.ds(ebase, epw)], src_v,
                                  sem_ld).wait()
        else:
            @pl.when(wid != part_w)
            def _():
                pltpu.make_async_copy(src_hbm.at[pl.ds(ebase, epw)], src_v,
                                      sem_ld).wait()

            @pl.when(wid == part_w)
            def _():
                pltpu.make_async_copy(
                    src_hbm.at[pl.ds(ebase, part_kc * _CH)],
                    src_v.at[pl.ds(0, part_kc * _CH)], sem_ld).wait()

        def _gather(k, b):
            return pltpu.make_async_copy(
                x_hbm.at[src_v.at[pl.ds(k * _CH, _CH)]], rows[b], sem_g[b])

        def _scatter(k, b):
            return pltpu.make_async_copy(
                rows[b], agg_sh.at[dst_v.at[b]], sem_s[b])

        def _wd_loads(k, b):
            cid = wid * k_chunks + k
            return [
                pltpu.make_async_copy(
                    w_hbm.at[pl.ds(cid * _CH, _CH)], w_v.at[b], sem_wd[b]),
                pltpu.make_async_copy(dst_hbm.at[cid], dst_v.at[b],
                                      sem_wd[b]),
            ]

        def _scale(k, b):
            def srow(g, _):
                wv = w_v[b, pl.ds(g * _LANES, _LANES)]
                for e in range(_LANES):
                    i = g * _LANES + e
                    wgt = wv[e]
                    for j in range(d // _LANES):
                        sl = pl.ds(j * _LANES, _LANES)
                        rows[b][i, sl] = rows[b][i, sl] * wgt
                return _

            lax.fori_loop(0, _CH // _LANES, srow, None)

        def outer(t, _):
            for b in range(2):
                k = 2 * t + b

                @pl.when(k >= 1)
                def _():
                    _scatter(k - 1, 1 - b).wait()

                @pl.when(k < kc_w - 1)
                def _():
                    _gather(k + 1, 1 - b).start()
                    for cp in _wd_loads(k + 1, 1 - b):
                        cp.start()

                _gather(k, b).wait()
                for cp in _wd_loads(k, b):
                    cp.wait()
                _scale(k, b)
                _scatter(k, b).start(add=True)
            return _

        _gather(0, 0).start()
        for cp in _wd_loads(0, 0):
            cp.start()
        lax.fori_loop(0, kc_w // 2, outer, None)
        _scatter(0, 1).wait()
        plsc.subcore_barrier()
        scope_edges.__exit__(None, None, None)

        scope_out = jax.named_scope("agg_writeout")
        scope_out.__enter__()
        obase = s * per_sub

        def _writeout(nrows):
            done = 0
            for step in _chunk_sizes(nrows, _CH):
                pltpu.sync_copy(agg_sh.at[pl.ds(obase + done, step)],
                                out_hbm.at[c, pl.ds(obase + done, step)])
                done += step

        @pl.when(s == _NS - 1)
        def _():
            _writeout(last_sub)

        @pl.when(s != _NS - 1)
        def _():
            _writeout(per_sub)

        scope_out.__exit__(None, None, None)

    return spmm


def _mlp_body(x_ref, agg_ref, w1_ref, b1_ref, g_ref, be_ref, w2_ref, b2_ref,
              o_ref):
    x = x_ref[...]
    h = (1.0 + _EPS1) * x + agg_ref[0] + agg_ref[1]
    z = lax.dot_general(h, w1_ref[...], (((1,), (1,)), ((), ())),
                        preferred_element_type=jnp.float32) + b1_ref[...]
    mean = jnp.mean(z, axis=0, keepdims=True)
    var = jnp.mean((z - mean) ** 2, axis=0, keepdims=True)
    zn = (z - mean) * lax.rsqrt(var + _BN_EPS) * g_ref[...] + be_ref[...]
    r = jnp.maximum(zn, 0.0)
    o_ref[...] = lax.dot_general(r, w2_ref[...], (((1,), (1,)), ((), ())),
                                 preferred_element_type=jnp.float32) + b2_ref[...]


@functools.lru_cache(maxsize=None)
def _make_mlp(n, d, h_dim, o_dim):
    return pl.pallas_call(
        _mlp_body,
        out_shape=jax.ShapeDtypeStruct((n, o_dim), jnp.float32),
    )


def kernel(x, edge_index, edge_weight, W1, b1, gamma, beta, W2, b2):
    n, d = x.shape
    e = edge_weight.shape[0]
    pad = (-e) % _CH
    dst = edge_index[0]
    src = edge_index[1]
    w = edge_weight
    if pad:
        fill = jnp.arange(pad, dtype=jnp.int32) % n
        src = jnp.concatenate([src, fill])
        dst = jnp.concatenate([dst, fill])
        w = jnp.concatenate([w, jnp.zeros((pad,), jnp.float32)])
    dst2d = dst.reshape((e + pad) // 128, 128)

    agg = _make_spmm(n, d, e + pad)(x, src, dst2d, w)

    mlp = _make_mlp(n, d, W1.shape[0], W2.shape[0])
    return mlp(x, agg, W1, b1.reshape(1, -1), gamma.reshape(1, -1),
               beta.reshape(1, -1), W2, b2.reshape(1, -1))


# speedup vs baseline: 3.0832x; 1.0026x over previous
import functools

import jax
import jax.numpy as jnp
from jax import lax
from jax.experimental import pallas as pl
from jax.experimental.pallas import tpu as pltpu
from jax.experimental.pallas import tpu_sc as plsc

_EPS1 = 1e-09
_BN_EPS = 1e-05

_NC = 2
_NS = 16
_NW = _NC * _NS
_CH = 128
_LANES = 16


def _chunk_sizes(total, step):
    out = []
    done = 0
    while done < total:
        out.append(min(step, total - done))
        done += step
    return out


@functools.lru_cache(maxsize=None)
def _make_spmm(n, d, e):
    per_sub = (((n + _NS - 1) // _NS) + 7) // 8 * 8
    last_sub = n - per_sub * (_NS - 1)
    assert last_sub > 0 and per_sub % 8 == 0 and last_sub % 8 == 0
    assert _CH == 128 and e % _CH == 0

    total_chunks = e // _CH
    kc_full = ((total_chunks + _NW - 1) // _NW + 1) // 2 * 2
    kcs = [max(0, min(kc_full, total_chunks - w * kc_full)) for w in range(_NW)]
    assert all(k >= 2 and k % 2 == 0 for k in kcs)
    part_w = min((w for w in range(_NW) if kcs[w] < kc_full), default=None)
    part_kc = kcs[part_w] if part_w is not None else kc_full
    k_chunks = kc_full
    epw = kc_full * _CH

    mesh = plsc.VectorSubcoreMesh(core_axis_name="c", subcore_axis_name="s")

    @functools.partial(
        pl.kernel,
        mesh=mesh,
        out_type=jax.ShapeDtypeStruct((_NC, n, d), jnp.float32),
        scratch_types=[
            pltpu.VMEM((epw,), jnp.int32),
            pltpu.VMEM((2, 128), jnp.int32),
            pltpu.VMEM((2, 128), jnp.float32),
            pltpu.VMEM((_CH, d), jnp.float32),
            pltpu.VMEM((_CH, d), jnp.float32),
            pltpu.VMEM_SHARED((n, d), jnp.float32),
            pltpu.SemaphoreType.DMA,
            pltpu.SemaphoreType.DMA,
            pltpu.SemaphoreType.DMA,
            pltpu.SemaphoreType.DMA,
            pltpu.SemaphoreType.DMA,
            pltpu.SemaphoreType.DMA,
            pltpu.SemaphoreType.DMA,
        ],
    )
    def spmm(x_hbm, src_hbm, dst_hbm, w_hbm, out_hbm,
             src_v, dst_v, w_v, rows0, rows1, agg_sh,
             sem_ld, sem_g0, sem_g1, sem_s0, sem_s1, sem_w0, sem_w1):
        c = lax.axis_index("c")
        s = lax.axis_index("s")
        wid = s * _NC + c
        rows = (rows0, rows1)
        sem_g = (sem_g0, sem_g1)
        sem_s = (sem_s0, sem_s1)
        sem_wd = (sem_w0, sem_w1)

        kc_w = jnp.clip(total_chunks - wid * kc_full, 0, kc_full)
        ebase = wid * epw
        if part_w is None:
            pltpu.async_copy(src_hbm.at[pl.ds(ebase, epw)], src_v, sem_ld)
        else:
            @pl.when(wid != part_w)
            def _():
                pltpu.async_copy(src_hbm.at[pl.ds(ebase, epw)], src_v,
                                 sem_ld)

            @pl.when(wid == part_w)
            def _():
                pltpu.async_copy(
                    src_hbm.at[pl.ds(ebase, part_kc * _CH)],
                    src_v.at[pl.ds(0, part_kc * _CH)], sem_ld)

        scope_zero = jax.named_scope("agg_zero")
        scope_zero.__enter__()
        zero = jnp.zeros((_LANES,), jnp.float32)

        def zbody(t, _):
            i = t // (d // _LANES)
            j = t % (d // _LANES)
            rows0[i, pl.ds(j * _LANES, _LANES)] = zero
            return _

        lax.fori_loop(0, _CH * (d // _LANES), zbody, None)

        zbase = s * per_sub

        def _blit_zeros(nrows):
            cps = []
            done = 0
            for step in _chunk_sizes(nrows, _CH):
                cps.append(pltpu.async_copy(
                    rows0.at[pl.ds(0, step)],
                    agg_sh.at[pl.ds(zbase + done, step)], sem_g0))
                done += step
            for cp in cps:
                cp.wait()

        @pl.when(s == _NS - 1)
        def _():
            _blit_zeros(last_sub)

        @pl.when(s != _NS - 1)
        def _():
            _blit_zeros(per_sub)

        plsc.subcore_barrier()
        scope_zero.__exit__(None, None, None)

        scope_edges = jax.named_scope("edge_loop")
        scope_edges.__enter__()
        if part_w is None:
            pltpu.make_async_copy(src_hbm.at[pl.ds(ebase, epw)], src_v,
                                  sem_ld).wait()
        else:
            @pl.when(wid != part_w)
            def _():
                pltpu.make_async_copy(src_hbm.at[pl.ds(ebase, epw)], src_v,
                                      sem_ld).wait()

            @pl.when(wid == part_w)
            def _():
                pltpu.make_async_copy(
                    src_hbm.at[pl.ds(ebase, part_kc * _CH)],
                    src_v.at[pl.ds(0, part_kc * _CH)], sem_ld).wait()

        def _gather(k, b):
            return pltpu.make_async_copy(
                x_hbm.at[src_v.at[pl.ds(k * _CH, _CH)]], rows[b], sem_g[b])

        def _scatter(k, b):
            return pltpu.make_async_copy(
                rows[b], agg_sh.at[dst_v.at[b]], sem_s[b])

        def _wd_loads(k, b):
            cid = wid * k_chunks + k
            return [
                pltpu.make_async_copy(
                    w_hbm.at[pl.ds(cid * _CH, _CH)], w_v.at[b], sem_wd[b]),
                pltpu.make_async_copy(dst_hbm.at[cid], dst_v.at[b],
                                      sem_wd[b]),
            ]

        def _scale(k, b):
            def srow(g, _):
                wv = w_v[b, pl.ds(g * _LANES, _LANES)]
                for e in range(_LANES):
                    i = g * _LANES + e
                    wgt = wv[e]
                    for j in range(d // _LANES):
                        sl = pl.ds(j * _LANES, _LANES)
                        rows[b][i, sl] = rows[b][i, sl] * wgt
                return _

            lax.fori_loop(0, _CH // _LANES, srow, None)

        def outer(t, _):
            for b in range(2):
                k = 2 * t + b

                @pl.when(k >= 1)
                def _():
                    _scatter(k - 1, 1 - b).wait()

                @pl.when(k < kc_w - 1)
                def _():
                    _gather(k + 1, 1 - b).start()
                    for cp in _wd_loads(k + 1, 1 - b):
                        cp.start()

                _gather(k, b).wait()
                for cp in _wd_loads(k, b):
                    cp.wait()
                _scale(k, b)
                _scatter(k, b).start(add=True)
            return _

        _gather(0, 0).start()
        for cp in _wd_loads(0, 0):
            cp.start()
        lax.fori_loop(0, kc_w // 2, outer, None)
        _scatter(0, 1).wait()
        plsc.subcore_barrier()
        scope_edges.__exit__(None, None, None)

        scope_out = jax.named_scope("agg_writeout")
        scope_out.__enter__()
        obase = s * per_sub

        def _writeout(nrows):
            done = 0
            for step in _chunk_sizes(nrows, _CH):
                pltpu.sync_copy(agg_sh.at[pl.ds(obase + done, step)],
                                out_hbm.at[c, pl.ds(obase + done, step)])
                done += step

        @pl.when(s == _NS - 1)
        def _():
            _writeout(last_sub)

        @pl.when(s != _NS - 1)
        def _():
            _writeout(per_sub)

        scope_out.__exit__(None, None, None)

    return spmm


def _mlp_body(x_ref, agg_ref, w1_ref, b1_ref, g_ref, be_ref, w2_ref, b2_ref,
              o_ref):
    x = x_ref[...]
    h = (1.0 + _EPS1) * x + agg_ref[0] + agg_ref[1]
    z = lax.dot_general(h, w1_ref[...], (((1,), (1,)), ((), ())),
                        preferred_element_type=jnp.float32) + b1_ref[...]
    mean = jnp.mean(z, axis=0, keepdims=True)
    var = jnp.mean((z - mean) ** 2, axis=0, keepdims=True)
    zn = (z - mean) * lax.rsqrt(var + _BN_EPS) * g_ref[...] + be_ref[...]
    r = jnp.maximum(zn, 0.0)
    o_ref[...] = lax.dot_general(r, w2_ref[...], (((1,), (1,)), ((), ())),
                                 preferred_element_type=jnp.float32) + b2_ref[...]


@functools.lru_cache(maxsize=None)
def _make_mlp(n, d, h_dim, o_dim):
    return pl.pallas_call(
        _mlp_body,
        out_shape=jax.ShapeDtypeStruct((n, o_dim), jnp.float32),
    )


def kernel(x, edge_index, edge_weight, W1, b1, gamma, beta, W2, b2):
    n, d = x.shape
    e = edge_weight.shape[0]
    pad = (-e) % _CH
    dst = edge_index[0]
    src = edge_index[1]
    w = edge_weight
    if pad:
        fill = jnp.arange(pad, dtype=jnp.int32) % n
        src = jnp.concatenate([src, fill])
        dst = jnp.concatenate([dst, fill])
        w = jnp.concatenate([w, jnp.zeros((pad,), jnp.float32)])
    dst2d = dst.reshape((e + pad) // 128, 128)

    agg = _make_spmm(n, d, e + pad)(x, src, dst2d, w)

    mlp = _make_mlp(n, d, W1.shape[0], W2.shape[0])
    return mlp(x, agg, W1, b1.reshape(1, -1), gamma.reshape(1, -1),
               beta.reshape(1, -1), W2, b2.reshape(1, -1))
